# baseline (speedup 1.0000x reference)
import functools

import jax
import jax.numpy as jnp
from jax import lax
from jax.experimental import pallas as pl
from jax.experimental.pallas import tpu as pltpu
from jax.experimental.pallas import tpu_sc as plsc

N = 10000
E = 320000
H = 128
IN_DIM = 128
NL = 4

NC = 2
NS = 16
NW = NC * NS
L = 16

CHUNK = 128
NCH0 = 56
NCH1 = 104
NCHD = 88
TCH = NS * (NCH0 + NCH1)
TCHP = NW * NCHD
EPAD = TCHP * CHUNK
NPAD = 10112
RPT = NPAD // NS

_PREC = jax.lax.Precision.DEFAULT

_mesh = plsc.VectorSubcoreMesh(core_axis_name="c", subcore_axis_name="s")


DEGW = 128


@functools.partial(
    pl.kernel,
    out_type=(
        jax.ShapeDtypeStruct((NC, NPAD, DEGW), jnp.float32),
        jax.ShapeDtypeStruct((NC, NPAD, DEGW), jnp.float32),
    ),
    mesh=_mesh,
    scratch_types=[
        pltpu.VMEM((NCHD, CHUNK), jnp.int32),
        pltpu.VMEM((CHUNK, DEGW), jnp.float32),
        pltpu.VMEM_SHARED((NPAD, DEGW), jnp.float32),
    ],
)
def _sc_degree(src_hbm, dst_hbm, ones_hbm, zeros_hbm, do_out, di_out,
               idx, ones_v, deg_sh):
    cid = lax.axis_index("c")
    sid = lax.axis_index("s")
    wid = cid * NS + sid
    pltpu.sync_copy(ones_hbm, ones_v)
    for edges_hbm, out_hbm in ((src_hbm, do_out), (dst_hbm, di_out)):
        pltpu.sync_copy(zeros_hbm.at[pl.ds(sid * RPT, RPT)],
                        deg_sh.at[pl.ds(sid * RPT, RPT)])
        pltpu.sync_copy(edges_hbm.at[wid], idx)
        plsc.subcore_barrier()

        def body(j, c):
            pltpu.sync_copy(ones_v, deg_sh.at[idx.at[j]], add=True)
            return c

        lax.fori_loop(0, NCHD, body, 0)
        plsc.subcore_barrier()
        pltpu.sync_copy(deg_sh.at[pl.ds(sid * RPT, RPT)],
                        out_hbm.at[cid].at[pl.ds(sid * RPT, RPT)])


@functools.partial(
    pl.kernel,
    out_type=jax.ShapeDtypeStruct((NC, NPAD, H), jnp.float32),
    mesh=_mesh,
    scratch_types=[
        pltpu.VMEM((max(NCH0, NCH1), CHUNK), jnp.int32),
        pltpu.VMEM((2, CHUNK), jnp.int32),
        pltpu.VMEM((2, CHUNK, H), jnp.float32),
        pltpu.VMEM_SHARED((NPAD, H), jnp.float32),
        pltpu.SemaphoreType.DMA,
        pltpu.SemaphoreType.DMA,
    ],
)
def _sc_agg(xs_hbm, src_hbm, dst_hbm, zeros_hbm, out_hbm,
            sidx, didx, rows, agg_sh, gsem, isem):
    cid = lax.axis_index("c")
    sid = lax.axis_index("s")
    base = jnp.where(cid == 0, sid * NCH0, NS * NCH0 + sid * NCH1)
    nch_w = jnp.where(cid == 0, NCH0, NCH1)
    pltpu.sync_copy(zeros_hbm.at[pl.ds(sid * RPT, RPT)],
                    agg_sh.at[pl.ds(sid * RPT, RPT)])
    pltpu.sync_copy(src_hbm.at[pl.ds(base, max(NCH0, NCH1))], sidx)
    plsc.subcore_barrier()

    def load_didx(j, b):
        pltpu.async_copy(dst_hbm.at[base + j], didx.at[b], isem)

    def wait_didx(j, b):
        pltpu.make_async_copy(dst_hbm.at[base + j], didx.at[b], isem).wait()

    def start_gather(j, b):
        pltpu.async_copy(xs_hbm.at[sidx.at[j]], rows.at[b], gsem)

    def wait_gather(j, b):
        pltpu.make_async_copy(xs_hbm.at[sidx.at[j]], rows.at[b], gsem).wait()

    def scatter(j, b):
        pltpu.sync_copy(rows.at[b], agg_sh.at[didx.at[b]], add=True)

    load_didx(0, 0)
    load_didx(1, 1)
    start_gather(0, 0)

    def body(jj, c):
        j = 2 * jj
        start_gather(j + 1, 1)
        wait_didx(j, 0)
        wait_gather(j, 0)
        scatter(j, 0)
        load_didx(j + 2, 0)
        start_gather(j + 2, 0)
        wait_didx(j + 1, 1)
        wait_gather(j + 1, 1)
        scatter(j + 1, 1)
        load_didx(j + 3, 1)
        return c

    lax.fori_loop(0, nch_w // 2 - 1, body, 0)
    j = nch_w - 2
    start_gather(j + 1, 1)
    wait_didx(j, 0)
    wait_gather(j, 0)
    scatter(j, 0)
    wait_didx(j + 1, 1)
    wait_gather(j + 1, 1)
    scatter(j + 1, 1)

    plsc.subcore_barrier()
    pltpu.sync_copy(agg_sh.at[pl.ds(sid * RPT, RPT)],
                    out_hbm.at[cid].at[pl.ds(sid * RPT, RPT)])


_PB = 2000


def _prep_body(do_ref, di_ref, h_ref, emb_ref, w_ref, nsnd_ref, x_ref,
               xsp_ref):
    dout = do_ref[0, :, 0:1] + do_ref[1, :, 0:1]
    din = di_ref[0, :, 0:1] + di_ref[1, :, 0:1]
    ns = jnp.where(dout > 0.0, lax.rsqrt(jnp.maximum(dout, 1.0)), 0.0)
    nd = jnp.where(din > 0.0, lax.rsqrt(jnp.maximum(din, 1.0)), 0.0)
    nsnd_ref[...] = jnp.concatenate([ns, nd], axis=1)
    iota = lax.broadcasted_iota(jnp.int32, (_PB, IN_DIM), 1)
    onehot = jnp.where(iota == h_ref[...], 1.0, 0.0).astype(jnp.float32)
    x = jnp.dot(onehot, emb_ref[...], precision=_PREC,
                preferred_element_type=jnp.float32)
    x_ref[...] = x
    xsp_ref[...] = jnp.dot(x, w_ref[...], precision=_PREC,
                           preferred_element_type=jnp.float32) * ns


def _tc_prep(dop, dip, h2, emb, w):
    return pl.pallas_call(
        _prep_body,
        grid=(N // _PB,),
        in_specs=[
            pl.BlockSpec((NC, _PB, DEGW), lambda i: (0, i, 0)),
            pl.BlockSpec((NC, _PB, DEGW), lambda i: (0, i, 0)),
            pl.BlockSpec((_PB, 1), lambda i: (i, 0)),
            pl.BlockSpec((IN_DIM, H), lambda i: (0, 0)),
            pl.BlockSpec((H, H), lambda i: (0, 0)),
        ],
        out_specs=(
            pl.BlockSpec((_PB, 2), lambda i: (i, 0)),
            pl.BlockSpec((_PB, H), lambda i: (i, 0)),
            pl.BlockSpec((_PB, H), lambda i: (i, 0)),
        ),
        out_shape=(
            jax.ShapeDtypeStruct((N, 2), jnp.float32),
            jax.ShapeDtypeStruct((N, H), jnp.float32),
            jax.ShapeDtypeStruct((NPAD, H), jnp.float32),
        ),
    )(dop, dip, h2, emb, w)


def _bn_relu_res(x, parts_ref, nsnd_ref, b_ref, g_ref, bt_ref):
    agg = parts_ref[0, :N, :] + parts_ref[1, :N, :]
    hgc = agg * nsnd_ref[:, 1:2] + b_ref[...]
    mu = jnp.mean(hgc, axis=0, keepdims=True)
    var = jnp.mean((hgc - mu) ** 2, axis=0, keepdims=True)
    hbn = (hgc - mu) * lax.rsqrt(var + 1e-5) * g_ref[...] + bt_ref[...]
    return x + jnp.maximum(hbn, 0.0)


def _mid_body(x_ref, parts_ref, nsnd_ref, b_ref, g_ref, bt_ref, w_ref,
              xn_ref, xsp_ref):
    xn = _bn_relu_res(x_ref[...], parts_ref, nsnd_ref, b_ref, g_ref, bt_ref)
    xn_ref[...] = xn
    xsp_ref[:N, :] = jnp.dot(xn, w_ref[...], precision=_PREC,
                             preferred_element_type=jnp.float32) \
        * nsnd_ref[:, 0:1]


def _tc_mid(x, parts, nsnd, b, g, bt, w):
    return pl.pallas_call(
        _mid_body,
        out_shape=(
            jax.ShapeDtypeStruct((N, H), jnp.float32),
            jax.ShapeDtypeStruct((NPAD, H), jnp.float32),
        ),
    )(x, parts, nsnd, b, g, bt, w)


def _final_body(x_ref, parts_ref, nsnd_ref, b_ref, g_ref, bt_ref,
                w0_ref, b0_ref, w1_ref, b1_ref, w2_ref, b2_ref, y_ref):
    xn = _bn_relu_res(x_ref[...], parts_ref, nsnd_ref, b_ref, g_ref, bt_ref)
    y = jnp.dot(xn, w0_ref[...], precision=_PREC,
                preferred_element_type=jnp.float32) + b0_ref[...]
    y = jnp.maximum(y, 0.0)
    y = jnp.dot(y, w1_ref[...], precision=_PREC,
                preferred_element_type=jnp.float32) + b1_ref[...]
    y = jnp.maximum(y, 0.0)
    y_ref[...] = jnp.dot(y, w2_ref[...], precision=_PREC,
                         preferred_element_type=jnp.float32) + b2_ref[...]


def _tc_final(x, parts, nsnd, b, g, bt, w0, b0, w1, b1, w2, b2):
    return pl.pallas_call(
        _final_body,
        out_shape=jax.ShapeDtypeStruct((N, 1), jnp.float32),
    )(x, parts, nsnd, b, g, bt, w0, b0, w1, b1, w2, b2)


def kernel(edge_index, h, e, emb, Ws, bs, gammas, betas, W0, b0, W1, b1, W2, b2):
    del e
    pad = jnp.full((EPAD - E,), N, jnp.int32)
    src_p = jnp.concatenate([edge_index[0], pad]).reshape(TCHP, CHUNK)
    dst_p = jnp.concatenate([edge_index[1], pad]).reshape(TCHP, CHUNK)
    src_d = src_p.reshape(NW, NCHD, CHUNK)
    dst_d = dst_p.reshape(NW, NCHD, CHUNK)
    zeros_big = jnp.zeros((NPAD, H), jnp.float32)
    ones_chunk = jnp.ones((CHUNK, DEGW), jnp.float32)
    zeros_deg = jnp.zeros((NPAD, DEGW), jnp.float32)

    dop, dip = _sc_degree(src_d, dst_d, ones_chunk, zeros_deg)
    nsnd, x, xsp = _tc_prep(dop, dip, h.reshape(N, 1), emb, Ws[0])

    for i in range(NL):
        parts = _sc_agg(xsp, src_p, dst_p, zeros_big)
        if i < NL - 1:
            x, xsp = _tc_mid(x, parts, nsnd, bs[i].reshape(1, H),
                             gammas[i].reshape(1, H), betas[i].reshape(1, H),
                             Ws[i + 1])
        else:
            y = _tc_final(x, parts, nsnd, bs[i].reshape(1, H),
                          gammas[i].reshape(1, H), betas[i].reshape(1, H),
                          W0, b0.reshape(1, H // 2), W1, b1.reshape(1, H // 4),
                          W2, b2.reshape(1, 1))
    return y


# speedup vs baseline: 1.3527x; 1.3527x over previous
import functools

import jax
import jax.numpy as jnp
from jax import lax
from jax.experimental import pallas as pl
from jax.experimental.pallas import tpu as pltpu
from jax.experimental.pallas import tpu_sc as plsc

N = 10000
E = 320000
H = 128
IN_DIM = 128
NL = 4

NC = 2
NS = 16
NW = NC * NS

CHUNK = 128
NCH = 79
EW = NCH * CHUNK
EPAD = NW * EW
NPAD = 10112
RPT = NPAD // NS
DEGW = 128

_PREC = jax.lax.Precision.DEFAULT

_mesh = plsc.VectorSubcoreMesh(core_axis_name="c", subcore_axis_name="s")


@functools.partial(
    pl.kernel,
    out_type=(
        jax.ShapeDtypeStruct((NC, NPAD, DEGW), jnp.float32),
        jax.ShapeDtypeStruct((NC, NPAD, DEGW), jnp.float32),
    ),
    mesh=_mesh,
    scratch_types=[
        pltpu.VMEM((CHUNK,), jnp.int32),
        pltpu.VMEM((CHUNK, DEGW), jnp.float32),
        pltpu.VMEM_SHARED((NPAD, DEGW), jnp.float32),
    ],
)
def _sc_degree(src_hbm, dst_hbm, ones_hbm, zeros_hbm, do_out, di_out,
               idx, ones_v, deg_sh):
    cid = lax.axis_index("c")
    sid = lax.axis_index("s")
    wid = cid * NS + sid
    base = wid * EW
    pltpu.sync_copy(ones_hbm, ones_v)
    for edges_hbm, out_hbm in ((src_hbm, do_out), (dst_hbm, di_out)):
        pltpu.sync_copy(zeros_hbm.at[pl.ds(sid * RPT, RPT)],
                        deg_sh.at[pl.ds(sid * RPT, RPT)])
        plsc.subcore_barrier()

        def body(j, c):
            off = base + j * CHUNK
            pltpu.sync_copy(edges_hbm.at[pl.ds(off, CHUNK)], idx)
            pltpu.sync_copy(ones_v, deg_sh.at[idx], add=True)
            return c

        lax.fori_loop(0, NCH, body, 0)
        plsc.subcore_barrier()
        pltpu.sync_copy(deg_sh.at[pl.ds(sid * RPT, RPT)],
                        out_hbm.at[cid].at[pl.ds(sid * RPT, RPT)])


@functools.partial(
    pl.kernel,
    out_type=jax.ShapeDtypeStruct((NC, NPAD, H), jnp.float32),
    mesh=_mesh,
    scratch_types=[
        pltpu.VMEM((CHUNK,), jnp.int32),
        pltpu.VMEM((CHUNK,), jnp.int32),
        pltpu.VMEM((CHUNK, H), jnp.float32),
        pltpu.VMEM_SHARED((NPAD, H), jnp.float32),
        pltpu.SemaphoreType.DMA,
    ],
)
def _sc_agg(xs_hbm, src_hbm, dst_hbm, zeros_hbm, out_hbm,
            sidx, didx, rows, agg_sh, sem):
    cid = lax.axis_index("c")
    sid = lax.axis_index("s")
    wid = cid * NS + sid
    pltpu.sync_copy(zeros_hbm.at[pl.ds(sid * RPT, RPT)],
                    agg_sh.at[pl.ds(sid * RPT, RPT)])
    plsc.subcore_barrier()
    base = wid * EW

    def body(j, c):
        off = base + j * CHUNK
        pltpu.sync_copy(src_hbm.at[pl.ds(off, CHUNK)], sidx)
        pltpu.sync_copy(dst_hbm.at[pl.ds(off, CHUNK)], didx)
        pltpu.async_copy(xs_hbm.at[sidx], rows, sem).wait()
        pltpu.sync_copy(rows, agg_sh.at[didx], add=True)
        return c

    lax.fori_loop(0, NCH, body, 0)
    plsc.subcore_barrier()
    pltpu.sync_copy(agg_sh.at[pl.ds(sid * RPT, RPT)],
                    out_hbm.at[cid].at[pl.ds(sid * RPT, RPT)])


_PB = 2000


def _prep_body(do_ref, di_ref, h_ref, emb_ref, ns_ref, nd_ref, x_ref):
    dout = do_ref[0, :, 0:1] + do_ref[1, :, 0:1]
    din = di_ref[0, :, 0:1] + di_ref[1, :, 0:1]
    ns_ref[...] = jnp.where(dout > 0.0, lax.rsqrt(jnp.maximum(dout, 1.0)), 0.0)
    nd_ref[...] = jnp.where(din > 0.0, lax.rsqrt(jnp.maximum(din, 1.0)), 0.0)
    iota = lax.broadcasted_iota(jnp.int32, (_PB, IN_DIM), 1)
    onehot = jnp.where(iota == h_ref[...], 1.0, 0.0).astype(jnp.float32)
    x_ref[...] = jnp.dot(onehot, emb_ref[...], precision=_PREC,
                         preferred_element_type=jnp.float32)


def _tc_prep(dop, dip, h2, emb):
    return pl.pallas_call(
        _prep_body,
        grid=(N // _PB,),
        in_specs=[
            pl.BlockSpec((NC, _PB, DEGW), lambda i: (0, i, 0)),
            pl.BlockSpec((NC, _PB, DEGW), lambda i: (0, i, 0)),
            pl.BlockSpec((_PB, 1), lambda i: (i, 0)),
            pl.BlockSpec((IN_DIM, H), lambda i: (0, 0)),
        ],
        out_specs=(
            pl.BlockSpec((_PB, 1), lambda i: (i, 0)),
            pl.BlockSpec((_PB, 1), lambda i: (i, 0)),
            pl.BlockSpec((_PB, H), lambda i: (i, 0)),
        ),
        out_shape=(
            jax.ShapeDtypeStruct((N, 1), jnp.float32),
            jax.ShapeDtypeStruct((N, 1), jnp.float32),
            jax.ShapeDtypeStruct((N, H), jnp.float32),
        ),
    )(dop, dip, h2, emb)


def _pre_body(x_ref, w_ref, ns_ref, out_ref):
    xs = jnp.dot(x_ref[...], w_ref[...], precision=_PREC,
                 preferred_element_type=jnp.float32) * ns_ref[...]
    out_ref[:N, :] = xs
    out_ref[N:, :] = jnp.zeros((NPAD - N, H), jnp.float32)


def _tc_pre(x, w, ns):
    return pl.pallas_call(
        _pre_body,
        out_shape=jax.ShapeDtypeStruct((NPAD, H), jnp.float32),
    )(x, w, ns)


def _post_body(x_ref, parts_ref, nd_ref, b_ref, g_ref, bt_ref, out_ref):
    agg = parts_ref[0, :N, :] + parts_ref[1, :N, :]
    hgc = agg * nd_ref[...] + b_ref[...]
    mu = jnp.mean(hgc, axis=0, keepdims=True)
    var = jnp.mean((hgc - mu) ** 2, axis=0, keepdims=True)
    hbn = (hgc - mu) * lax.rsqrt(var + 1e-5) * g_ref[...] + bt_ref[...]
    out_ref[...] = x_ref[...] + jnp.maximum(hbn, 0.0)


def _tc_post(x, parts, nd, b, g, bt):
    return pl.pallas_call(
        _post_body,
        out_shape=jax.ShapeDtypeStruct((N, H), jnp.float32),
    )(x, parts, nd, b, g, bt)


def _mlp_body(x_ref, w0_ref, b0_ref, w1_ref, b1_ref, w2_ref, b2_ref, y_ref):
    y = jnp.dot(x_ref[...], w0_ref[...], precision=_PREC,
                preferred_element_type=jnp.float32) + b0_ref[...]
    y = jnp.maximum(y, 0.0)
    y = jnp.dot(y, w1_ref[...], precision=_PREC,
                preferred_element_type=jnp.float32) + b1_ref[...]
    y = jnp.maximum(y, 0.0)
    y_ref[...] = jnp.dot(y, w2_ref[...], precision=_PREC,
                         preferred_element_type=jnp.float32) + b2_ref[...]


def _tc_mlp(x, w0, b0, w1, b1, w2, b2):
    return pl.pallas_call(
        _mlp_body,
        out_shape=jax.ShapeDtypeStruct((N, 1), jnp.float32),
    )(x, w0, b0, w1, b1, w2, b2)


def kernel(edge_index, h, e, emb, Ws, bs, gammas, betas, W0, b0, W1, b1, W2, b2):
    del e
    pad = jnp.full((EPAD - E,), N, jnp.int32)
    src_p = jnp.concatenate([edge_index[0], pad])
    dst_p = jnp.concatenate([edge_index[1], pad])

    ones_chunk = jnp.ones((CHUNK, DEGW), jnp.float32)
    zeros_deg = jnp.zeros((NPAD, DEGW), jnp.float32)
    zeros_big = jnp.zeros((NPAD, H), jnp.float32)

    dop, dip = _sc_degree(src_p, dst_p, ones_chunk, zeros_deg)
    ns, nd, x = _tc_prep(dop, dip, h.reshape(N, 1), emb)

    for i in range(NL):
        xsp = _tc_pre(x, Ws[i], ns)
        parts = _sc_agg(xsp, src_p, dst_p, zeros_big)
        x = _tc_post(x, parts, nd, bs[i].reshape(1, H),
                     gammas[i].reshape(1, H), betas[i].reshape(1, H))

    return _tc_mlp(x, W0, b0.reshape(1, H // 2), W1, b1.reshape(1, H // 4),
                   W2, b2.reshape(1, 1))


# speedup vs baseline: 1.4233x; 1.0522x over previous
import functools

import jax
import jax.numpy as jnp
from jax import lax
from jax.experimental import pallas as pl
from jax.experimental.pallas import tpu as pltpu
from jax.experimental.pallas import tpu_sc as plsc

N = 10000
E = 320000
H = 128
IN_DIM = 128
NL = 4

NC = 2
NS = 16
NW = NC * NS

CHUNK = 128
NCH = 79
EW = NCH * CHUNK
EPAD = NW * EW
NPAD = 10112
RPT = NPAD // NS
DEGW = 128

_PREC = jax.lax.Precision.DEFAULT

_mesh = plsc.VectorSubcoreMesh(core_axis_name="c", subcore_axis_name="s")


@functools.partial(
    pl.kernel,
    out_type=(
        jax.ShapeDtypeStruct((NC, NPAD, DEGW), jnp.float32),
        jax.ShapeDtypeStruct((NC, NPAD, DEGW), jnp.float32),
    ),
    mesh=_mesh,
    scratch_types=[
        pltpu.VMEM((NCH, CHUNK), jnp.int32),
        pltpu.VMEM((CHUNK, DEGW), jnp.float32),
        pltpu.VMEM_SHARED((NPAD, DEGW), jnp.float32),
    ],
)
def _sc_degree(src_hbm, dst_hbm, ones_hbm, zeros_hbm, do_out, di_out,
               idx, ones_v, deg_sh):
    cid = lax.axis_index("c")
    sid = lax.axis_index("s")
    wid = cid * NS + sid
    pltpu.sync_copy(ones_hbm, ones_v)
    for edges_hbm, out_hbm in ((src_hbm, do_out), (dst_hbm, di_out)):
        pltpu.sync_copy(zeros_hbm.at[pl.ds(sid * RPT, RPT)],
                        deg_sh.at[pl.ds(sid * RPT, RPT)])
        pltpu.sync_copy(edges_hbm.at[wid], idx)
        plsc.subcore_barrier()

        def body(j, c):
            pltpu.sync_copy(ones_v, deg_sh.at[idx.at[j]], add=True)
            return c

        lax.fori_loop(0, NCH, body, 0)
        plsc.subcore_barrier()
        pltpu.sync_copy(deg_sh.at[pl.ds(sid * RPT, RPT)],
                        out_hbm.at[cid].at[pl.ds(sid * RPT, RPT)])


@functools.partial(
    pl.kernel,
    out_type=jax.ShapeDtypeStruct((NC, NPAD, H), jnp.float32),
    mesh=_mesh,
    scratch_types=[
        pltpu.VMEM((CHUNK,), jnp.int32),
        pltpu.VMEM((CHUNK,), jnp.int32),
        pltpu.VMEM((CHUNK, H), jnp.float32),
        pltpu.VMEM_SHARED((NPAD, H), jnp.float32),
        pltpu.SemaphoreType.DMA,
    ],
)
def _sc_agg(xs_hbm, src_hbm, dst_hbm, zeros_hbm, out_hbm,
            sidx, didx, rows, agg_sh, sem):
    cid = lax.axis_index("c")
    sid = lax.axis_index("s")
    wid = cid * NS + sid
    pltpu.sync_copy(zeros_hbm.at[pl.ds(sid * RPT, RPT)],
                    agg_sh.at[pl.ds(sid * RPT, RPT)])
    plsc.subcore_barrier()
    base = wid * EW

    def body(j, c):
        off = base + j * CHUNK
        pltpu.sync_copy(src_hbm.at[pl.ds(off, CHUNK)], sidx)
        pltpu.sync_copy(dst_hbm.at[pl.ds(off, CHUNK)], didx)
        pltpu.async_copy(xs_hbm.at[sidx], rows, sem).wait()
        pltpu.sync_copy(rows, agg_sh.at[didx], add=True)
        return c

    lax.fori_loop(0, NCH, body, 0)
    plsc.subcore_barrier()
    pltpu.sync_copy(agg_sh.at[pl.ds(sid * RPT, RPT)],
                    out_hbm.at[cid].at[pl.ds(sid * RPT, RPT)])


_PB = 2000


def _prep_body(do_ref, di_ref, h_ref, emb_ref, ns_ref, nd_ref, x_ref):
    dout = do_ref[0, :, 0:1] + do_ref[1, :, 0:1]
    din = di_ref[0, :, 0:1] + di_ref[1, :, 0:1]
    ns_ref[...] = jnp.where(dout > 0.0, lax.rsqrt(jnp.maximum(dout, 1.0)), 0.0)
    nd_ref[...] = jnp.where(din > 0.0, lax.rsqrt(jnp.maximum(din, 1.0)), 0.0)
    iota = lax.broadcasted_iota(jnp.int32, (_PB, IN_DIM), 1)
    onehot = jnp.where(iota == h_ref[...], 1.0, 0.0).astype(jnp.float32)
    x_ref[...] = jnp.dot(onehot, emb_ref[...], precision=_PREC,
                         preferred_element_type=jnp.float32)


def _tc_prep(dop, dip, h2, emb):
    return pl.pallas_call(
        _prep_body,
        grid=(N // _PB,),
        in_specs=[
            pl.BlockSpec((NC, _PB, DEGW), lambda i: (0, i, 0)),
            pl.BlockSpec((NC, _PB, DEGW), lambda i: (0, i, 0)),
            pl.BlockSpec((_PB, 1), lambda i: (i, 0)),
            pl.BlockSpec((IN_DIM, H), lambda i: (0, 0)),
        ],
        out_specs=(
            pl.BlockSpec((_PB, 1), lambda i: (i, 0)),
            pl.BlockSpec((_PB, 1), lambda i: (i, 0)),
            pl.BlockSpec((_PB, H), lambda i: (i, 0)),
        ),
        out_shape=(
            jax.ShapeDtypeStruct((N, 1), jnp.float32),
            jax.ShapeDtypeStruct((N, 1), jnp.float32),
            jax.ShapeDtypeStruct((N, H), jnp.float32),
        ),
    )(dop, dip, h2, emb)


def _pre_body(x_ref, w_ref, ns_ref, out_ref):
    xs = jnp.dot(x_ref[...], w_ref[...], precision=_PREC,
                 preferred_element_type=jnp.float32) * ns_ref[...]
    out_ref[:N, :] = xs
    out_ref[N:, :] = jnp.zeros((NPAD - N, H), jnp.float32)


def _tc_pre(x, w, ns):
    return pl.pallas_call(
        _pre_body,
        out_shape=jax.ShapeDtypeStruct((NPAD, H), jnp.float32),
    )(x, w, ns)


def _post_body(x_ref, parts_ref, nd_ref, b_ref, g_ref, bt_ref, out_ref):
    agg = parts_ref[0, :N, :] + parts_ref[1, :N, :]
    hgc = agg * nd_ref[...] + b_ref[...]
    mu = jnp.mean(hgc, axis=0, keepdims=True)
    var = jnp.mean((hgc - mu) ** 2, axis=0, keepdims=True)
    hbn = (hgc - mu) * lax.rsqrt(var + 1e-5) * g_ref[...] + bt_ref[...]
    out_ref[...] = x_ref[...] + jnp.maximum(hbn, 0.0)


def _tc_post(x, parts, nd, b, g, bt):
    return pl.pallas_call(
        _post_body,
        out_shape=jax.ShapeDtypeStruct((N, H), jnp.float32),
    )(x, parts, nd, b, g, bt)


def _mlp_body(x_ref, w0_ref, b0_ref, w1_ref, b1_ref, w2_ref, b2_ref, y_ref):
    y = jnp.dot(x_ref[...], w0_ref[...], precision=_PREC,
                preferred_element_type=jnp.float32) + b0_ref[...]
    y = jnp.maximum(y, 0.0)
    y = jnp.dot(y, w1_ref[...], precision=_PREC,
                preferred_element_type=jnp.float32) + b1_ref[...]
    y = jnp.maximum(y, 0.0)
    y_ref[...] = jnp.dot(y, w2_ref[...], precision=_PREC,
                         preferred_element_type=jnp.float32) + b2_ref[...]


def _tc_mlp(x, w0, b0, w1, b1, w2, b2):
    return pl.pallas_call(
        _mlp_body,
        out_shape=jax.ShapeDtypeStruct((N, 1), jnp.float32),
    )(x, w0, b0, w1, b1, w2, b2)


def kernel(edge_index, h, e, emb, Ws, bs, gammas, betas, W0, b0, W1, b1, W2, b2):
    del e
    pad = jnp.full((EPAD - E,), N, jnp.int32)
    src_p = jnp.concatenate([edge_index[0], pad])
    dst_p = jnp.concatenate([edge_index[1], pad])

    ones_chunk = jnp.ones((CHUNK, DEGW), jnp.float32)
    zeros_deg = jnp.zeros((NPAD, DEGW), jnp.float32)
    zeros_big = jnp.zeros((NPAD, H), jnp.float32)

    dop, dip = _sc_degree(src_p.reshape(NW, NCH, CHUNK),
                          dst_p.reshape(NW, NCH, CHUNK), ones_chunk, zeros_deg)
    ns, nd, x = _tc_prep(dop, dip, h.reshape(N, 1), emb)

    for i in range(NL):
        xsp = _tc_pre(x, Ws[i], ns)
        parts = _sc_agg(xsp, src_p, dst_p, zeros_big)
        x = _tc_post(x, parts, nd, bs[i].reshape(1, H),
                     gammas[i].reshape(1, H), betas[i].reshape(1, H))

    return _tc_mlp(x, W0, b0.reshape(1, H // 2), W1, b1.reshape(1, H // 4),
                   W2, b2.reshape(1, 1))
